# baseline (speedup 1.0000x reference)
import jax
import jax.numpy as jnp
from jax import lax
from jax.experimental import pallas as pl
from jax.experimental.pallas import tpu as pltpu
from jax.experimental.pallas import tpu_sc as plsc

LEADS = 16384
L = 4096
HALF = 2048
NC = 2
NS = 16
NW = NC * NS
RPW = LEADS // NW
RB = 8
NBATCH = RPW // RB
SLOT = RB * L

def _fix_rows(st_v, buf_v, rbase, soff, h):
    rbase = rbase + h * (RB // 4)
    soff = soff + h * (RB // 4)
    jv = lax.broadcasted_iota(jnp.int32, (16,), 0)
    z16 = jnp.zeros((16,), jnp.float32)

    def row(i, _):
        s = st_v[pl.ds(rbase + i, 16)][0]
        ap = s & (-16)
        rmd = s - ap
        br = soff + i
        for j in range(1, HALF // 16):
            off = pl.multiple_of((ap + 16 * j) & (L - 1), 16)
            buf_v[br, pl.ds(off, 16)] = z16
        o1 = pl.multiple_of(ap & (L - 1), 16)
        w1 = buf_v[br, pl.ds(o1, 16)]
        buf_v[br, pl.ds(o1, 16)] = jnp.where(jv < rmd, w1, 0.0)
        o2 = pl.multiple_of((ap + HALF) & (L - 1), 16)
        w2 = buf_v[br, pl.ds(o2, 16)]
        buf_v[br, pl.ds(o2, 16)] = jnp.where(jv >= rmd, w2, 0.0)
        return 0

    lax.fori_loop(0, RB // 4, row, 0)


def _sc_body(x_hbm, st_hbm, o_hbm, st_v, buf_v, sem_r0, sem_r1, sem_r2,
             sem_w0, sem_w1, sem_w2):
    wid = lax.axis_index("s") * NC + lax.axis_index("c")
    base = wid * RPW
    pltpu.sync_copy(st_hbm.at[pl.ds(base, RPW)], st_v.at[pl.ds(0, RPW)])

    sem_r = (sem_r0, sem_r1, sem_r2)
    sem_w = (sem_w0, sem_w1, sem_w2)

    def rd(bi, slot):
        return pltpu.make_async_copy(
            x_hbm.at[pl.ds(base + bi * RB, RB), :],
            buf_v.at[pl.ds(slot * RB, RB), :], sem_r[slot])

    def wr_half(bi, slot, h):
        hh = RB // 4
        return pltpu.make_async_copy(
            buf_v.at[pl.ds(slot * RB + h * hh, hh), :],
            o_hbm.at[pl.ds(base + bi * RB + h * hh, hh), :], sem_w[slot])

    rd(0, 0).start()
    rd(1, 1).start()

    def triple(p, _):
        for k in range(3):
            b = 3 * p + k
            ks = (k + 2) % 3

            @pl.when(b + 2 < NBATCH)
            def _():
                @pl.when(b >= 1)
                def _():
                    for h in range(4):
                        wr_half(b - 1, ks, h).wait()
                rd(b + 2, ks).start()

            rd(b, k).wait()
            for h in range(4):
                _fix_rows(st_v, buf_v, b * RB, k * RB, h)
                wr_half(b, k, h).start()
        return 0

    lax.fori_loop(0, (NBATCH - 1) // 3, triple, 0)

    rd(NBATCH - 1, 0).wait()
    for h in range(4):
        _fix_rows(st_v, buf_v, (NBATCH - 1) * RB, 0, h)
        wr_half(NBATCH - 1, 0, h).start()

    for bi, sl in ((NBATCH - 3, 1), (NBATCH - 2, 2), (NBATCH - 1, 0)):
        for h in range(4):
            wr_half(bi, sl, h).wait()


def kernel(x, starts):
    out = pl.kernel(
        _sc_body,
        out_type=jax.ShapeDtypeStruct((LEADS, L), jnp.float32),
        mesh=plsc.VectorSubcoreMesh(core_axis_name="c", subcore_axis_name="s"),
        scratch_types=[
            pltpu.VMEM((RPW + 16,), jnp.int32),
            pltpu.VMEM((3 * RB, L), jnp.float32),
            pltpu.SemaphoreType.DMA,
            pltpu.SemaphoreType.DMA,
            pltpu.SemaphoreType.DMA,
            pltpu.SemaphoreType.DMA,
            pltpu.SemaphoreType.DMA,
            pltpu.SemaphoreType.DMA,
        ],
    )(x, starts)
    return out


# speedup vs baseline: 1.0599x; 1.0599x over previous
import jax
import jax.numpy as jnp
from jax import lax
from jax.experimental import pallas as pl
from jax.experimental.pallas import tpu as pltpu
from jax.experimental.pallas import tpu_sc as plsc

LEADS = 16384
L = 4096
HALF = 2048
NC = 2
NS = 16
NW = NC * NS
RPW = LEADS // NW
RB = 8
NBATCH = RPW // RB
SLOT = RB * L

def _fix_rows(st_v, buf_v, rbase, soff, h):
    rbase = rbase + h * (RB // 2)
    soff = soff + h * (RB // 2)
    jv = lax.broadcasted_iota(jnp.int32, (16,), 0)
    z16 = jnp.zeros((16,), jnp.float32)

    def row(i, _):
        s = st_v[pl.ds(rbase + i, 16)][0]
        ap = s & (-16)
        rmd = s - ap
        br = soff + i
        for j in range(1, HALF // 16):
            off = pl.multiple_of((ap + 16 * j) & (L - 1), 16)
            buf_v[br, pl.ds(off, 16)] = z16
        o1 = pl.multiple_of(ap & (L - 1), 16)
        w1 = buf_v[br, pl.ds(o1, 16)]
        buf_v[br, pl.ds(o1, 16)] = jnp.where(jv < rmd, w1, 0.0)
        o2 = pl.multiple_of((ap + HALF) & (L - 1), 16)
        w2 = buf_v[br, pl.ds(o2, 16)]
        buf_v[br, pl.ds(o2, 16)] = jnp.where(jv >= rmd, w2, 0.0)
        return 0

    lax.fori_loop(0, RB // 2, row, 0)


def _sc_body(x_hbm, st_hbm, o_hbm, st_v, buf_v, sem_r0, sem_r1, sem_r2,
             sem_w0, sem_w1, sem_w2):
    wid = lax.axis_index("s") * NC + lax.axis_index("c")
    base = wid * RPW
    pltpu.sync_copy(st_hbm.at[pl.ds(base, RPW)], st_v.at[pl.ds(0, RPW)])

    sem_r = (sem_r0, sem_r1, sem_r2)
    sem_w = (sem_w0, sem_w1, sem_w2)

    def rd(bi, slot):
        return pltpu.make_async_copy(
            x_hbm.at[pl.ds(base + bi * RB, RB), :],
            buf_v.at[pl.ds(slot * RB, RB), :], sem_r[slot])

    def wr_half(bi, slot, h):
        hh = RB // 2
        return pltpu.make_async_copy(
            buf_v.at[pl.ds(slot * RB + h * hh, hh), :],
            o_hbm.at[pl.ds(base + bi * RB + h * hh, hh), :], sem_w[slot])

    rd(0, 0).start()
    rd(1, 1).start()

    def triple(p, _):
        for k in range(3):
            b = 3 * p + k
            ks = (k + 2) % 3

            rd(b, k).wait()
            _fix_rows(st_v, buf_v, b * RB, k * RB, 0)
            wr_half(b, k, 0).start()

            @pl.when(b + 2 < NBATCH)
            def _():
                @pl.when(b >= 1)
                def _():
                    wr_half(b - 1, ks, 0).wait()
                    wr_half(b - 1, ks, 1).wait()
                rd(b + 2, ks).start()

            _fix_rows(st_v, buf_v, b * RB, k * RB, 1)
            wr_half(b, k, 1).start()
        return 0

    lax.fori_loop(0, (NBATCH - 1) // 3, triple, 0)

    rd(NBATCH - 1, 0).wait()
    _fix_rows(st_v, buf_v, (NBATCH - 1) * RB, 0, 0)
    wr_half(NBATCH - 1, 0, 0).start()
    _fix_rows(st_v, buf_v, (NBATCH - 1) * RB, 0, 1)
    wr_half(NBATCH - 1, 0, 1).start()

    for bi, sl in ((NBATCH - 3, 1), (NBATCH - 2, 2), (NBATCH - 1, 0)):
        wr_half(bi, sl, 0).wait()
        wr_half(bi, sl, 1).wait()


def kernel(x, starts):
    out = pl.kernel(
        _sc_body,
        out_type=jax.ShapeDtypeStruct((LEADS, L), jnp.float32),
        mesh=plsc.VectorSubcoreMesh(core_axis_name="c", subcore_axis_name="s"),
        scratch_types=[
            pltpu.VMEM((RPW + 16,), jnp.int32),
            pltpu.VMEM((3 * RB, L), jnp.float32),
            pltpu.SemaphoreType.DMA,
            pltpu.SemaphoreType.DMA,
            pltpu.SemaphoreType.DMA,
            pltpu.SemaphoreType.DMA,
            pltpu.SemaphoreType.DMA,
            pltpu.SemaphoreType.DMA,
        ],
    )(x, starts)
    return out


# speedup vs baseline: 1.0674x; 1.0070x over previous
import jax
import jax.numpy as jnp
from jax import lax
from jax.experimental import pallas as pl
from jax.experimental.pallas import tpu as pltpu
from jax.experimental.pallas import tpu_sc as plsc

LEADS = 16384
L = 4096
HALF = 2048
NC = 2
NS = 16
NW = NC * NS
RPW = LEADS // NW
RB = 8
NBATCH = RPW // RB
SLOT = RB * L

def _fix_rows(st_v, buf_v, rbase, soff, h):
    rbase = rbase + h * (RB // 2)
    soff = soff + h * (RB // 2)
    jv = lax.broadcasted_iota(jnp.int32, (16,), 0)
    z16 = jnp.zeros((16,), jnp.float32)

    def row(i, _):
        s = st_v[pl.ds(rbase + i, 16)][0]
        ap = s & (-16)
        rmd = s - ap
        br = soff + i
        for j in range(1, HALF // 16):
            off = pl.multiple_of((ap + 16 * j) & (L - 1), 16)
            buf_v[br, pl.ds(off, 16)] = z16
        o1 = pl.multiple_of(ap & (L - 1), 16)
        w1 = buf_v[br, pl.ds(o1, 16)]
        buf_v[br, pl.ds(o1, 16)] = jnp.where(jv < rmd, w1, 0.0)
        o2 = pl.multiple_of((ap + HALF) & (L - 1), 16)
        w2 = buf_v[br, pl.ds(o2, 16)]
        buf_v[br, pl.ds(o2, 16)] = jnp.where(jv >= rmd, w2, 0.0)
        return 0

    lax.fori_loop(0, RB // 2, row, 0)


def _sc_body(x_hbm, st_hbm, o_hbm, st_v, buf_v, sem_r0, sem_r1, sem_r2,
             sem_w0, sem_w1, sem_w2):
    wid = lax.axis_index("s") * NC + lax.axis_index("c")
    base = wid * RPW
    pltpu.sync_copy(st_hbm.at[pl.ds(base, RPW)], st_v.at[pl.ds(0, RPW)])

    sem_r = (sem_r0, sem_r1, sem_r2)
    sem_w = (sem_w0, sem_w1, sem_w2)

    def rd(bi, slot):
        return pltpu.make_async_copy(
            x_hbm.at[pl.ds(base + bi * RB, RB), :],
            buf_v.at[pl.ds(slot * RB, RB), :], sem_r[slot])

    def wr_half(bi, slot, h):
        hh = RB // 2
        return pltpu.make_async_copy(
            buf_v.at[pl.ds(slot * RB + h * hh, hh), :],
            o_hbm.at[pl.ds(base + bi * RB + h * hh, hh), :], sem_w[slot])

    rd(0, 0).start()
    rd(1, 1).start()

    def triple(p, _):
        for k in range(3):
            b = 3 * p + k
            ks = (k + 2) % 3

            @pl.when(b + 2 < NBATCH)
            def _():
                @pl.when(b >= 1)
                def _():
                    wr_half(b - 1, ks, 0).wait()
                    wr_half(b - 1, ks, 1).wait()
                rd(b + 2, ks).start()

            rd(b, k).wait()
            _fix_rows(st_v, buf_v, b * RB, k * RB, 0)
            wr_half(b, k, 0).start()
            _fix_rows(st_v, buf_v, b * RB, k * RB, 1)
            wr_half(b, k, 1).start()
        return 0

    lax.fori_loop(0, (NBATCH - 1) // 3, triple, 0)

    rd(NBATCH - 1, 0).wait()
    _fix_rows(st_v, buf_v, (NBATCH - 1) * RB, 0, 0)
    wr_half(NBATCH - 1, 0, 0).start()
    _fix_rows(st_v, buf_v, (NBATCH - 1) * RB, 0, 1)
    wr_half(NBATCH - 1, 0, 1).start()

    for bi, sl in ((NBATCH - 3, 1), (NBATCH - 2, 2), (NBATCH - 1, 0)):
        wr_half(bi, sl, 0).wait()
        wr_half(bi, sl, 1).wait()


def kernel(x, starts):
    out = pl.kernel(
        _sc_body,
        out_type=jax.ShapeDtypeStruct((LEADS, L), jnp.float32),
        mesh=plsc.VectorSubcoreMesh(core_axis_name="c", subcore_axis_name="s"),
        scratch_types=[
            pltpu.VMEM((RPW + 16,), jnp.int32),
            pltpu.VMEM((3 * RB, L), jnp.float32),
            pltpu.SemaphoreType.DMA,
            pltpu.SemaphoreType.DMA,
            pltpu.SemaphoreType.DMA,
            pltpu.SemaphoreType.DMA,
            pltpu.SemaphoreType.DMA,
            pltpu.SemaphoreType.DMA,
        ],
    )(x, starts)
    return out
